# baseline (speedup 1.0000x reference)
import functools

import jax
import jax.numpy as jnp
from jax import lax
from jax.experimental import pallas as pl
from jax.experimental.pallas import tpu as pltpu
from jax.experimental.pallas import tpu_sc as plsc

NC, NS = 2, 16
NW = NC * NS
BATCH, SEQ, D = 4, 2048, 512
WPB = NW // BATCH
BPW = SEQ // WPB
CHUNK = 32
NCHUNK = BPW // CHUNK
NBUF = 7

_mesh = plsc.VectorSubcoreMesh(core_axis_name="c", subcore_axis_name="s")


@functools.partial(
    pl.kernel,
    mesh=_mesh,
    out_type=jax.ShapeDtypeStruct((BATCH, SEQ, D), jnp.float32),
    scratch_types=[
        pltpu.VMEM((BPW,), jnp.int32),
        *[pltpu.VMEM((CHUNK, D), jnp.float32) for _ in range(NBUF)],
        *[pltpu.SemaphoreType.DMA for _ in range(2 * NBUF)],
    ],
)
def _embed_gather(idx_hbm, table_hbm, out_hbm, idx_v, *scratch):
    bufs = scratch[:NBUF]
    gsems = scratch[NBUF:2 * NBUF]
    osems = scratch[2 * NBUF:]

    wid = lax.axis_index("s") * NC + lax.axis_index("c")
    row = wid // WPB
    seq0 = (wid % WPB) * BPW
    pltpu.sync_copy(idx_hbm.at[row, pl.ds(seq0, BPW)], idx_v)

    def start_gather(c):
        b = c % NBUF
        return pltpu.async_copy(
            table_hbm.at[idx_v.at[pl.ds(c * CHUNK, CHUNK)]], bufs[b], gsems[b])

    gh = [None] * NCHUNK
    oh = [None] * NCHUNK
    for c in range(min(NBUF, NCHUNK)):
        gh[c] = start_gather(c)
    for c in range(NCHUNK):
        b = c % NBUF
        gh[c].wait()
        oh[c] = pltpu.async_copy(
            bufs[b], out_hbm.at[row, pl.ds(seq0 + c * CHUNK, CHUNK)], osems[b])
        nxt = c + NBUF
        if nxt < NCHUNK:
            oh[c].wait()
            oh[c] = None
            gh[nxt] = start_gather(nxt)
    for c in range(NCHUNK):
        if oh[c] is not None:
            oh[c].wait()


def kernel(x, table):
    return _embed_gather(x.astype(jnp.int32), table)


# speedup vs baseline: 1.0238x; 1.0238x over previous
import functools

import jax
import jax.numpy as jnp
from jax import lax
from jax.experimental import pallas as pl
from jax.experimental.pallas import tpu as pltpu
from jax.experimental.pallas import tpu_sc as plsc

NC, NS = 2, 16
NW = NC * NS
BATCH, SEQ, D = 4, 2048, 512
WPB = NW // BATCH
BPW = SEQ // WPB
CHUNK = 32
NCHUNK = BPW // CHUNK
NBUF = 7

_mesh = plsc.VectorSubcoreMesh(core_axis_name="c", subcore_axis_name="s")


@functools.partial(
    pl.kernel,
    mesh=_mesh,
    out_type=jax.ShapeDtypeStruct((BATCH, SEQ, D), jnp.float32),
    scratch_types=[
        pltpu.VMEM((BPW,), jnp.int32),
        *[pltpu.VMEM((CHUNK, D), jnp.float32) for _ in range(NBUF)],
        *[pltpu.SemaphoreType.DMA for _ in range(2 * NBUF)],
    ],
)
def _embed_gather(idx_hbm, table_hbm, out_hbm, idx_v, *scratch):
    bufs = scratch[:NBUF]
    gsems = scratch[NBUF:2 * NBUF]
    osems = scratch[2 * NBUF:]

    wid = lax.axis_index("s") * NC + lax.axis_index("c")
    row = wid // WPB
    seq0 = (wid % WPB) * BPW
    pltpu.sync_copy(idx_hbm.at[row, pl.ds(seq0, BPW)], idx_v)

    def start_gather(c):
        b = c % NBUF
        return pltpu.async_copy(
            table_hbm.at[idx_v.at[pl.ds(c * CHUNK, CHUNK)]], bufs[b], gsems[b])

    gh = [None] * NCHUNK
    oh = [None] * NCHUNK
    for c in range(min(NBUF, NCHUNK)):
        gh[c] = start_gather(c)
    for c in range(NCHUNK):
        b = c % NBUF
        gh[c].wait()
        oh[c] = pltpu.async_copy(
            bufs[b], out_hbm.at[row, pl.ds(seq0 + c * CHUNK, CHUNK)], osems[b])
        d = c - 2
        if d >= 0 and d + NBUF < NCHUNK:
            oh[d].wait()
            oh[d] = None
            gh[d + NBUF] = start_gather(d + NBUF)
    for c in range(NCHUNK):
        if oh[c] is not None:
            oh[c].wait()


def kernel(x, table):
    return _embed_gather(x.astype(jnp.int32), table)


# speedup vs baseline: 1.0327x; 1.0087x over previous
import functools

import jax
import jax.numpy as jnp
from jax import lax
from jax.experimental import pallas as pl
from jax.experimental.pallas import tpu as pltpu
from jax.experimental.pallas import tpu_sc as plsc

NC, NS = 2, 16
NW = NC * NS
BATCH, SEQ, D = 4, 2048, 512
WPB = NW // BATCH
BPW = SEQ // WPB
CHUNK = 16
NCHUNK = BPW // CHUNK
NBUF = 14

_mesh = plsc.VectorSubcoreMesh(core_axis_name="c", subcore_axis_name="s")


@functools.partial(
    pl.kernel,
    mesh=_mesh,
    out_type=jax.ShapeDtypeStruct((BATCH, SEQ, D), jnp.float32),
    scratch_types=[
        pltpu.VMEM((BPW,), jnp.int32),
        *[pltpu.VMEM((CHUNK, D), jnp.float32) for _ in range(NBUF)],
        *[pltpu.SemaphoreType.DMA for _ in range(2 * NBUF)],
    ],
)
def _embed_gather(idx_hbm, table_hbm, out_hbm, idx_v, *scratch):
    bufs = scratch[:NBUF]
    gsems = scratch[NBUF:2 * NBUF]
    osems = scratch[2 * NBUF:]

    wid = lax.axis_index("s") * NC + lax.axis_index("c")
    row = wid // WPB
    seq0 = (wid % WPB) * BPW
    pltpu.sync_copy(idx_hbm.at[row, pl.ds(seq0, BPW)], idx_v)

    def start_gather(c):
        b = c % NBUF
        return pltpu.async_copy(
            table_hbm.at[idx_v.at[pl.ds(c * CHUNK, CHUNK)]], bufs[b], gsems[b])

    gh = [None] * NCHUNK
    oh = [None] * NCHUNK
    for c in range(min(NBUF, NCHUNK)):
        gh[c] = start_gather(c)
    for c in range(NCHUNK):
        b = c % NBUF
        gh[c].wait()
        oh[c] = pltpu.async_copy(
            bufs[b], out_hbm.at[row, pl.ds(seq0 + c * CHUNK, CHUNK)], osems[b])
        d = c - 2
        if d >= 0 and d + NBUF < NCHUNK:
            oh[d].wait()
            oh[d] = None
            gh[d + NBUF] = start_gather(d + NBUF)
    for c in range(NCHUNK):
        if oh[c] is not None:
            oh[c].wait()


def kernel(x, table):
    return _embed_gather(x.astype(jnp.int32), table)


# speedup vs baseline: 1.0406x; 1.0077x over previous
import functools

import jax
import jax.numpy as jnp
from jax import lax
from jax.experimental import pallas as pl
from jax.experimental.pallas import tpu as pltpu
from jax.experimental.pallas import tpu_sc as plsc

NC, NS = 2, 16
NW = NC * NS
BATCH, SEQ, D = 4, 2048, 512
WPB = NW // BATCH
BPW = SEQ // WPB
CHUNK = 16
NCHUNK = BPW // CHUNK
NBUF = 14

_mesh = plsc.VectorSubcoreMesh(core_axis_name="c", subcore_axis_name="s")


@functools.partial(
    pl.kernel,
    mesh=_mesh,
    out_type=jax.ShapeDtypeStruct((BATCH, SEQ, D), jnp.float32),
    scratch_types=[
        pltpu.VMEM((BPW,), jnp.int32),
        *[pltpu.VMEM((CHUNK, D), jnp.float32) for _ in range(NBUF)],
        *[pltpu.SemaphoreType.DMA for _ in range(2 * NBUF + 1)],
    ],
)
def _embed_gather(idx_hbm, table_hbm, out_hbm, idx_v, *scratch):
    bufs = scratch[:NBUF]
    gsems = scratch[NBUF:2 * NBUF]
    osems = scratch[2 * NBUF:3 * NBUF]
    isem = scratch[3 * NBUF]

    wid = lax.axis_index("s") * NC + lax.axis_index("c")
    row = wid // WPB
    seq0 = (wid % WPB) * BPW

    HALF = BPW // 2
    ih0 = pltpu.async_copy(
        idx_hbm.at[row, pl.ds(seq0, HALF)], idx_v.at[pl.ds(0, HALF)], isem)
    ih1 = pltpu.async_copy(
        idx_hbm.at[row, pl.ds(seq0 + HALF, HALF)],
        idx_v.at[pl.ds(HALF, HALF)], isem)

    def start_gather(c):
        b = c % NBUF
        return pltpu.async_copy(
            table_hbm.at[idx_v.at[pl.ds(c * CHUNK, CHUNK)]], bufs[b], gsems[b])

    gh = [None] * NCHUNK
    oh = [None] * NCHUNK
    ih0.wait()
    for c in range(min(NBUF, NCHUNK)):
        if c * CHUNK == HALF:
            ih1.wait()
            ih1 = None
        gh[c] = start_gather(c)
    if ih1 is not None:
        ih1.wait()
    for c in range(NCHUNK):
        b = c % NBUF
        gh[c].wait()
        oh[c] = pltpu.async_copy(
            bufs[b], out_hbm.at[row, pl.ds(seq0 + c * CHUNK, CHUNK)], osems[b])
        d = c - 2
        if d >= 0 and d + NBUF < NCHUNK:
            oh[d].wait()
            oh[d] = None
            gh[d + NBUF] = start_gather(d + NBUF)
    for c in range(NCHUNK):
        if oh[c] is not None:
            oh[c].wait()


def kernel(x, table):
    return _embed_gather(x.astype(jnp.int32), table)
